# baseline (speedup 1.0000x reference)
import jax
import jax.numpy as jnp
from jax.experimental import pallas as pl
from jax.experimental.pallas import tpu as pltpu

_V = 1000


_S = 5


def _onehot_block(labt_ref, out_ref):
    b = labt_ref.shape[-1]
    j = jax.lax.rem(pl.program_id(0), _S)
    iota = jax.lax.broadcasted_iota(jnp.int32, (_V // _S, b), 0) + j * (_V // _S)
    out_ref[...] = (iota == labt_ref[0]).astype(jnp.float32)


def kernel(labels):
    if labels.ndim == 1:
        labels = labels.reshape(labels.shape[0], -1)
    b, f = labels.shape
    labt = labels.T.reshape(f, 1, b)
    out_t = pl.pallas_call(
        _onehot_block,
        grid=(f * _S,),
        in_specs=[pl.BlockSpec((1, 1, b), lambda i: (i // _S, 0, 0))],
        out_specs=pl.BlockSpec((_V // _S, b), lambda i: (i, 0)),
        out_shape=jax.ShapeDtypeStruct((f * _V, b), jnp.float32),
        compiler_params=pltpu.CompilerParams(
            dimension_semantics=("arbitrary",),
            vmem_limit_bytes=100 * 1024 * 1024,
        ),
    )(labt)
    return out_t.T


# speedup vs baseline: 1.0117x; 1.0117x over previous
import jax
import jax.numpy as jnp
from jax.experimental import pallas as pl
from jax.experimental.pallas import tpu as pltpu

_V = 1000


_M = 2


def _onehot_block(labt_ref, out_ref):
    b = labt_ref.shape[-1]
    iota = jax.lax.broadcasted_iota(jnp.int32, (_V, b), 0)
    for j in range(_M):
        out_ref[pl.ds(j * _V, _V), :] = (iota == labt_ref[j]).astype(
            jnp.float32
        )


def kernel(labels):
    if labels.ndim == 1:
        labels = labels.reshape(labels.shape[0], -1)
    b, f = labels.shape
    labt = labels.T.reshape(f, 1, b)
    out_t = pl.pallas_call(
        _onehot_block,
        grid=(f // _M,),
        in_specs=[pl.BlockSpec((_M, 1, b), lambda i: (i, 0, 0))],
        out_specs=pl.BlockSpec((_M * _V, b), lambda i: (i, 0)),
        out_shape=jax.ShapeDtypeStruct((f * _V, b), jnp.float32),
        compiler_params=pltpu.CompilerParams(
            dimension_semantics=("arbitrary",),
            vmem_limit_bytes=100 * 1024 * 1024,
        ),
    )(labt)
    return out_t.T


# speedup vs baseline: 1.0216x; 1.0098x over previous
import jax
import jax.numpy as jnp
from jax.experimental import pallas as pl
from jax.experimental.pallas import tpu as pltpu

_V = 1000


def _onehot_block(labt_ref, out_ref):
    b = labt_ref.shape[-1]
    iota = jax.lax.broadcasted_iota(jnp.int32, (_V, b), 0)
    out_ref[...] = (iota == labt_ref[0]).astype(jnp.float32)


def kernel(labels):
    if labels.ndim == 1:
        labels = labels.reshape(labels.shape[0], -1)
    b, f = labels.shape
    labt = labels.T.reshape(f, 1, b)
    out_t = pl.pallas_call(
        _onehot_block,
        grid=(f,),
        in_specs=[pl.BlockSpec((1, 1, b), lambda i: (i, 0, 0))],
        out_specs=pl.BlockSpec((_V, b), lambda i: (i, 0)),
        out_shape=jax.ShapeDtypeStruct((f * _V, b), jnp.float32),
        compiler_params=pltpu.CompilerParams(
            dimension_semantics=("arbitrary",),
            vmem_limit_bytes=100 * 1024 * 1024,
        ),
    )(labt)
    return out_t.T
